# baseline (speedup 1.0000x reference)
import math

import jax
import jax.numpy as jnp
from jax.experimental import pallas as pl

_B, _N = 64, 32
_D_IN, _MUL, _D_OUT = 8, 16, 8
_NB, _H = 3, 100
_MIN_R, _MAX_R = 0.0, 1.0
_STEP = (_MAX_R - _MIN_R) / (_NB - 1)
_BT = 16
_DIMS = [(_D_IN, _MUL), (_MUL, _MUL), (_MUL, _MUL), (_MUL, _D_OUT)]
_T = 128
_DMAX = 1.7330508
_DELTA = _DMAX / (_T - 1)


def _swish(x):
    return x * jax.nn.sigmoid(x)


def _body(x_ref, gi_ref, gj_ref, *rest):
    w_refs = rest[:-1]
    out_ref = rest[-1]
    bt, n = _BT, _N
    m = bt * n * n

    diff = gi_ref[...] - gj_ref[...]
    d2 = jnp.sum(diff * diff, axis=-1, keepdims=True)
    d = jnp.sqrt(d2 + 1e-12)
    xi = d.reshape(m, 1) * (1.0 / _DELTA)
    xi = jnp.minimum(xi, float(_T - 1))
    tt = (jax.lax.broadcasted_iota(jnp.int32, (1, _T), 1)
          .astype(jnp.float32))
    what = jnp.maximum(1.0 - jnp.abs(xi - tt), 0.0)

    dg = (jax.lax.broadcasted_iota(jnp.int32, (_T, 1), 0)
          .astype(jnp.float32) * _DELTA)
    centers = (jax.lax.broadcasted_iota(jnp.int32, (1, _NB), 1)
               .astype(jnp.float32) * _STEP + _MIN_R)
    tg = (dg - centers) * (1.0 / _STEP)
    basis_g = jnp.exp(-(tg * tg))

    x = x_ref[...]
    for li, (di, do) in enumerate(_DIMS):
        w0, b0, w1, b1, w2, b2 = w_refs[6 * li:6 * li + 6]
        uv = do * di
        hg = _swish(jnp.dot(basis_g, w0[...],
                            preferred_element_type=jnp.float32) + b0[...])
        hg = _swish(jnp.dot(hg, w1[...],
                            preferred_element_type=jnp.float32) + b1[...])
        tab = ((jnp.dot(hg, w2[...], preferred_element_type=jnp.float32)
                + b2[...])
               * (1.0 / (math.sqrt(di) * math.sqrt(n))))
        rw = jnp.dot(what, tab, preferred_element_type=jnp.float32)
        rw4 = rw.reshape(bt, n, n, uv)
        xt = jnp.concatenate([x] * do, axis=-1)
        y = jnp.sum(rw4 * xt[:, None, :, :], axis=2)
        pi = jax.lax.broadcasted_iota(jnp.int32, (uv, do), 0)
        ui = jax.lax.broadcasted_iota(jnp.int32, (uv, do), 1)
        e = (pi // di == ui).astype(jnp.float32)
        o = jnp.dot(y.reshape(bt * n, uv), e, preferred_element_type=jnp.float32)
        x = o.reshape(bt, n, do)
        if li < 3:
            x = _swish(x)
    out_ref[...] = x


def kernel(input, geometry, radial_params):
    gi = jnp.broadcast_to(geometry[:, :, None, :], (_B, _N, _N, 3))
    gj = jnp.broadcast_to(geometry[:, None, :, :], (_B, _N, _N, 3))
    gi = gi.reshape(_B, _N * _N, 3)
    gj = gj.reshape(_B, _N * _N, 3)
    args = [input, gi, gj]
    in_specs = [
        pl.BlockSpec((_BT, _N, _D_IN), lambda i: (i, 0, 0)),
        pl.BlockSpec((_BT, _N * _N, 3), lambda i: (i, 0, 0)),
        pl.BlockSpec((_BT, _N * _N, 3), lambda i: (i, 0, 0)),
    ]
    for p in radial_params:
        w0, b0, w1, b1, w2, b2 = p
        for a in (w0, b0.reshape(1, -1), w1, b1.reshape(1, -1),
                  w2, b2.reshape(1, -1)):
            args.append(a)
            in_specs.append(
                pl.BlockSpec(a.shape, lambda i, r=a.ndim: (0,) * r))
    out = pl.pallas_call(
        _body,
        grid=(_B // _BT,),
        in_specs=in_specs,
        out_specs=pl.BlockSpec((_BT, _N, _D_OUT), lambda i: (i, 0, 0)),
        out_shape=jax.ShapeDtypeStruct((_B, _N, _D_OUT), jnp.float32),
    )(*args)
    return out


# speedup vs baseline: 1.0029x; 1.0029x over previous
import math

import jax
import jax.numpy as jnp
from jax.experimental import pallas as pl

_B, _N = 64, 32
_D_IN, _MUL, _D_OUT = 8, 16, 8
_NB, _H = 3, 100
_MIN_R, _MAX_R = 0.0, 1.0
_STEP = (_MAX_R - _MIN_R) / (_NB - 1)
_BT = 8
_DIMS = [(_D_IN, _MUL), (_MUL, _MUL), (_MUL, _MUL), (_MUL, _D_OUT)]
_T = 64
_DMAX = 1.7330508
_DELTA = _DMAX / (_T - 1)


def _swish(x):
    return x * jax.nn.sigmoid(x)


def _body(x_ref, gi_ref, gj_ref, *rest):
    w_refs = rest[:-1]
    out_ref = rest[-1]
    bt, n = _BT, _N
    m = bt * n * n

    diff = gi_ref[...] - gj_ref[...]
    d2 = jnp.sum(diff * diff, axis=-1, keepdims=True)
    d = jnp.sqrt(d2 + 1e-12)
    xi = d.reshape(m, 1) * (1.0 / _DELTA)
    xi = jnp.minimum(xi, float(_T - 1))
    tt = (jax.lax.broadcasted_iota(jnp.int32, (1, _T), 1)
          .astype(jnp.float32))
    what = jnp.maximum(1.0 - jnp.abs(xi - tt), 0.0)

    dg = (jax.lax.broadcasted_iota(jnp.int32, (_T, 1), 0)
          .astype(jnp.float32) * _DELTA)
    centers = (jax.lax.broadcasted_iota(jnp.int32, (1, _NB), 1)
               .astype(jnp.float32) * _STEP + _MIN_R)
    tg = (dg - centers) * (1.0 / _STEP)
    basis_g = jnp.exp(-(tg * tg))

    x = x_ref[...]
    for li, (di, do) in enumerate(_DIMS):
        w0, b0, w1, b1, w2, b2 = w_refs[6 * li:6 * li + 6]
        uv = do * di
        hg = _swish(jnp.dot(basis_g, w0[...],
                            preferred_element_type=jnp.float32) + b0[...])
        hg = _swish(jnp.dot(hg, w1[...],
                            preferred_element_type=jnp.float32) + b1[...])
        tab = ((jnp.dot(hg, w2[...], preferred_element_type=jnp.float32)
                + b2[...])
               * (1.0 / (math.sqrt(di) * math.sqrt(n))))
        rw = jnp.dot(what, tab, preferred_element_type=jnp.float32)
        rw4 = rw.reshape(bt, n, n, uv)
        xt = jnp.concatenate([x] * do, axis=-1)
        y = jnp.sum(rw4 * xt[:, None, :, :], axis=2)
        pi = jax.lax.broadcasted_iota(jnp.int32, (uv, do), 0)
        ui = jax.lax.broadcasted_iota(jnp.int32, (uv, do), 1)
        e = (pi // di == ui).astype(jnp.float32)
        o = jnp.dot(y.reshape(bt * n, uv), e, preferred_element_type=jnp.float32)
        x = o.reshape(bt, n, do)
        if li < 3:
            x = _swish(x)
    out_ref[...] = x


def kernel(input, geometry, radial_params):
    gi = jnp.broadcast_to(geometry[:, :, None, :], (_B, _N, _N, 3))
    gj = jnp.broadcast_to(geometry[:, None, :, :], (_B, _N, _N, 3))
    gi = gi.reshape(_B, _N * _N, 3)
    gj = gj.reshape(_B, _N * _N, 3)
    args = [input, gi, gj]
    in_specs = [
        pl.BlockSpec((_BT, _N, _D_IN), lambda i: (i, 0, 0)),
        pl.BlockSpec((_BT, _N * _N, 3), lambda i: (i, 0, 0)),
        pl.BlockSpec((_BT, _N * _N, 3), lambda i: (i, 0, 0)),
    ]
    for p in radial_params:
        w0, b0, w1, b1, w2, b2 = p
        for a in (w0, b0.reshape(1, -1), w1, b1.reshape(1, -1),
                  w2, b2.reshape(1, -1)):
            args.append(a)
            in_specs.append(
                pl.BlockSpec(a.shape, lambda i, r=a.ndim: (0,) * r))
    out = pl.pallas_call(
        _body,
        grid=(_B // _BT,),
        in_specs=in_specs,
        out_specs=pl.BlockSpec((_BT, _N, _D_OUT), lambda i: (i, 0, 0)),
        out_shape=jax.ShapeDtypeStruct((_B, _N, _D_OUT), jnp.float32),
    )(*args)
    return out


# speedup vs baseline: 1.0113x; 1.0084x over previous
import math

import jax
import jax.numpy as jnp
from jax.experimental import pallas as pl

_B, _N = 64, 32
_D_IN, _MUL, _D_OUT = 8, 16, 8
_NB, _H = 3, 100
_MIN_R, _MAX_R = 0.0, 1.0
_STEP = (_MAX_R - _MIN_R) / (_NB - 1)
_BT = 8
_DIMS = [(_D_IN, _MUL), (_MUL, _MUL), (_MUL, _MUL), (_MUL, _D_OUT)]
_T = 128
_DMAX = 1.7330508
_DELTA = _DMAX / (_T - 1)


def _swish(x):
    return x * jax.nn.sigmoid(x)


def _body(x_ref, gi_ref, gj_ref, *rest):
    w_refs = rest[:-1]
    out_ref = rest[-1]
    bt, n = _BT, _N
    m = bt * n * n

    diff = gi_ref[...] - gj_ref[...]
    d2 = jnp.sum(diff * diff, axis=-1, keepdims=True)
    d = jnp.sqrt(d2 + 1e-12)
    xi = d.reshape(m, 1) * (1.0 / _DELTA)
    xi = jnp.minimum(xi, float(_T - 1))
    tt = (jax.lax.broadcasted_iota(jnp.int32, (1, _T), 1)
          .astype(jnp.float32))
    what = jnp.maximum(1.0 - jnp.abs(xi - tt), 0.0)

    dg = (jax.lax.broadcasted_iota(jnp.int32, (_T, 1), 0)
          .astype(jnp.float32) * _DELTA)
    centers = (jax.lax.broadcasted_iota(jnp.int32, (1, _NB), 1)
               .astype(jnp.float32) * _STEP + _MIN_R)
    tg = (dg - centers) * (1.0 / _STEP)
    basis_g = jnp.exp(-(tg * tg))

    x = x_ref[...]
    for li, (di, do) in enumerate(_DIMS):
        w0, b0, w1, b1, w2, b2 = w_refs[6 * li:6 * li + 6]
        uv = do * di
        hg = _swish(jnp.dot(basis_g, w0[...],
                            preferred_element_type=jnp.float32) + b0[...])
        hg = _swish(jnp.dot(hg, w1[...],
                            preferred_element_type=jnp.float32) + b1[...])
        tab = ((jnp.dot(hg, w2[...], preferred_element_type=jnp.float32)
                + b2[...])
               * (1.0 / (math.sqrt(di) * math.sqrt(n))))
        rw = jnp.dot(what, tab, preferred_element_type=jnp.float32)
        rw4 = rw.reshape(bt, n, n, uv)
        xt = jnp.concatenate([x] * do, axis=-1)
        y = jnp.sum(rw4 * xt[:, None, :, :], axis=2)
        pi = jax.lax.broadcasted_iota(jnp.int32, (uv, do), 0)
        ui = jax.lax.broadcasted_iota(jnp.int32, (uv, do), 1)
        e = (pi // di == ui).astype(jnp.float32)
        o = jnp.dot(y.reshape(bt * n, uv), e, preferred_element_type=jnp.float32)
        x = o.reshape(bt, n, do)
        if li < 3:
            x = _swish(x)
    out_ref[...] = x


def kernel(input, geometry, radial_params):
    gi = jnp.broadcast_to(geometry[:, :, None, :], (_B, _N, _N, 3))
    gj = jnp.broadcast_to(geometry[:, None, :, :], (_B, _N, _N, 3))
    gi = gi.reshape(_B, _N * _N, 3)
    gj = gj.reshape(_B, _N * _N, 3)
    args = [input, gi, gj]
    in_specs = [
        pl.BlockSpec((_BT, _N, _D_IN), lambda i: (i, 0, 0)),
        pl.BlockSpec((_BT, _N * _N, 3), lambda i: (i, 0, 0)),
        pl.BlockSpec((_BT, _N * _N, 3), lambda i: (i, 0, 0)),
    ]
    for p in radial_params:
        w0, b0, w1, b1, w2, b2 = p
        for a in (w0, b0.reshape(1, -1), w1, b1.reshape(1, -1),
                  w2, b2.reshape(1, -1)):
            args.append(a)
            in_specs.append(
                pl.BlockSpec(a.shape, lambda i, r=a.ndim: (0,) * r))
    out = pl.pallas_call(
        _body,
        grid=(_B // _BT,),
        in_specs=in_specs,
        out_specs=pl.BlockSpec((_BT, _N, _D_OUT), lambda i: (i, 0, 0)),
        out_shape=jax.ShapeDtypeStruct((_B, _N, _D_OUT), jnp.float32),
    )(*args)
    return out


# speedup vs baseline: 1.3252x; 1.3104x over previous
import math

import jax
import jax.numpy as jnp
from jax.experimental import pallas as pl

_B, _N = 64, 32
_D_IN, _MUL, _D_OUT = 8, 16, 8
_NB, _H = 3, 100
_MIN_R, _MAX_R = 0.0, 1.0
_STEP = (_MAX_R - _MIN_R) / (_NB - 1)
_BT = 8
_DIMS = [(_D_IN, _MUL), (_MUL, _MUL), (_MUL, _MUL), (_MUL, _D_OUT)]
_T = 128
_DMAX = 1.7330508
_DELTA = _DMAX / (_T - 1)


def _swish(x):
    return x * jax.nn.sigmoid(x)


def _body(x_ref, g_ref, *rest):
    w_refs = rest[:-1]
    out_ref = rest[-1]
    bt, n = _BT, _N
    m = bt * n * n

    g = g_ref[...]
    gi = jnp.broadcast_to(g[:, :, None, :], (bt, n, n, 3)).reshape(m, 3)
    gj = jnp.broadcast_to(g[:, None, :, :], (bt, n, n, 3)).reshape(m, 3)
    diff = gi - gj
    d2 = jnp.sum(diff * diff, axis=-1, keepdims=True)
    d = jnp.sqrt(d2 + 1e-12)
    xi = d * (1.0 / _DELTA)
    xi = jnp.minimum(xi, float(_T - 1))
    tt = (jax.lax.broadcasted_iota(jnp.int32, (1, _T), 1)
          .astype(jnp.float32))
    what = jnp.maximum(1.0 - jnp.abs(xi - tt), 0.0)

    dg = (jax.lax.broadcasted_iota(jnp.int32, (_T, 1), 0)
          .astype(jnp.float32) * _DELTA)
    centers = (jax.lax.broadcasted_iota(jnp.int32, (1, _NB), 1)
               .astype(jnp.float32) * _STEP + _MIN_R)
    tg = (dg - centers) * (1.0 / _STEP)
    basis_g = jnp.exp(-(tg * tg))

    x = x_ref[...]
    for li, (di, do) in enumerate(_DIMS):
        w0, b0, w1, b1, w2, b2 = w_refs[6 * li:6 * li + 6]
        uv = do * di
        hg = _swish(jnp.dot(basis_g, w0[...],
                            preferred_element_type=jnp.float32) + b0[...])
        hg = _swish(jnp.dot(hg, w1[...],
                            preferred_element_type=jnp.float32) + b1[...])
        tab = ((jnp.dot(hg, w2[...], preferred_element_type=jnp.float32)
                + b2[...])
               * (1.0 / (math.sqrt(di) * math.sqrt(n))))
        rw = jnp.dot(what, tab, preferred_element_type=jnp.float32)
        rw4 = rw.reshape(bt, n, n, uv)
        xt = jnp.concatenate([x] * do, axis=-1)
        y = jnp.sum(rw4 * xt[:, None, :, :], axis=2)
        pi = jax.lax.broadcasted_iota(jnp.int32, (uv, do), 0)
        ui = jax.lax.broadcasted_iota(jnp.int32, (uv, do), 1)
        e = (pi // di == ui).astype(jnp.float32)
        o = jnp.dot(y.reshape(bt * n, uv), e, preferred_element_type=jnp.float32)
        x = o.reshape(bt, n, do)
        if li < 3:
            x = _swish(x)
    out_ref[...] = x


def kernel(input, geometry, radial_params):
    args = [input, geometry]
    in_specs = [
        pl.BlockSpec((_BT, _N, _D_IN), lambda i: (i, 0, 0)),
        pl.BlockSpec((_BT, _N, 3), lambda i: (i, 0, 0)),
    ]
    for p in radial_params:
        w0, b0, w1, b1, w2, b2 = p
        for a in (w0, b0.reshape(1, -1), w1, b1.reshape(1, -1),
                  w2, b2.reshape(1, -1)):
            args.append(a)
            in_specs.append(
                pl.BlockSpec(a.shape, lambda i, r=a.ndim: (0,) * r))
    out = pl.pallas_call(
        _body,
        grid=(_B // _BT,),
        in_specs=in_specs,
        out_specs=pl.BlockSpec((_BT, _N, _D_OUT), lambda i: (i, 0, 0)),
        out_shape=jax.ShapeDtypeStruct((_B, _N, _D_OUT), jnp.float32),
    )(*args)
    return out
